# baseline (speedup 1.0000x reference)
import functools

import jax
import jax.numpy as jnp
from jax import lax
from jax.experimental import pallas as pl
from jax.experimental.pallas import tpu as pltpu
from jax.experimental.pallas import tpu_sc as plsc

NHID = 512
K_TOP = 8
ALPHA = 0.5

Q = 512
L = 512
EN = 8192
EL = 8192
D = 512
N_NODES = 10000

PAD = 8
EPAD = 12288
TI = 256
NG = EPAD // PAD
TJ = NG
NI = EPAD // TI
NJ = PAD
GPT_I = TI // PAD
QW = 48
RPAD = 576
BIG = 1e30
NEG = -1e9


def _split3(v):
    h = v.astype(jnp.bfloat16)
    r = v - h.astype(jnp.float32)
    m = r.astype(jnp.bfloat16)
    lo = (r - m.astype(jnp.float32)).astype(jnp.bfloat16)
    return h, m, lo


def _dot_t(a, b):
    return lax.dot_general(a, b, (((1,), (1,)), ((), ())),
                           preferred_element_type=jnp.float32)


def _dot(a, b):
    return lax.dot_general(a, b, (((1,), (0,)), ((), ())),
                           preferred_element_type=jnp.float32)


def _split_dot(v, sel):
    h, m, lo = _split3(v)
    return _dot(h, sel) + _dot(m, sel) + _dot(lo, sel)


def _split_dot_l(sel, v):
    h, m, lo = _split3(v)
    return _dot(sel, h) + _dot(sel, m) + _dot(sel, lo)


def _k1_body(xb_ref, wb_ref, o_ref):
    o_ref[...] = _dot(xb_ref[...], wb_ref[...])


def _matmul_h(xb, wb):
    return pl.pallas_call(
        _k1_body,
        grid=(5,),
        in_specs=[
            pl.BlockSpec((2000, D), lambda i: (i, 0)),
            pl.BlockSpec((D, NHID), lambda i: (0, 0)),
        ],
        out_specs=pl.BlockSpec((2000, NHID), lambda i: (i, 0)),
        out_shape=jax.ShapeDtypeStruct((10000, NHID), jnp.float32),
    )(xb, wb)


_GB = 50176


def _sc_gather(htab, idx):
    info = plsc.get_sparse_core_info()
    nw = info.num_cores * info.num_subcores
    b_per_w = _GB // nw
    chunk = 56
    nchunks = b_per_w // chunk
    mesh = plsc.VectorSubcoreMesh(core_axis_name="c", subcore_axis_name="s")

    @functools.partial(
        pl.kernel, mesh=mesh,
        out_type=jax.ShapeDtypeStruct((_GB, NHID), jnp.float32),
        scratch_types=[
            pltpu.VMEM((b_per_w,), jnp.int32),
            pltpu.VMEM((chunk, NHID), jnp.float32),
            pltpu.SemaphoreType.DMA,
        ],
    )
    def k(table_hbm, idx_hbm, out_hbm, idx_v, rows_v, sem):
        wid = lax.axis_index("s") * info.num_cores + lax.axis_index("c")
        base = wid * b_per_w
        pltpu.sync_copy(idx_hbm.at[pl.ds(base, b_per_w)], idx_v)

        @pl.loop(0, nchunks)
        def _chunked(c):
            pltpu.async_copy(table_hbm.at[idx_v.at[pl.ds(c * chunk, chunk)]],
                             rows_v, sem).wait()
            pltpu.sync_copy(rows_v, out_hbm.at[pl.ds(base + c * chunk, chunk)])

    return k(htab, idx)


def _k3_body(g0, g1, vmask, ef_o, n2_o):
    m = (g0[...] + g1[...]) * 0.5
    ef_o[...] = m.astype(jnp.bfloat16)
    n2 = jnp.sum(m * m, axis=1, keepdims=True)
    va = vmask[...][:, 0:1]
    n2_o[...] = jnp.broadcast_to(jnp.where(va > 0.5, n2, BIG), (1024, 8))


def _edge_prep(g0, g1, vmask):
    return pl.pallas_call(
        _k3_body,
        grid=(EPAD // 1024,),
        in_specs=[
            pl.BlockSpec((1024, D), lambda i: (i, 0)),
            pl.BlockSpec((1024, D), lambda i: (i, 0)),
            pl.BlockSpec((1024, 8), lambda i: (i, 0)),
        ],
        out_specs=[
            pl.BlockSpec((1024, D), lambda i: (i, 0)),
            pl.BlockSpec((1024, 8), lambda i: (i, 0)),
        ],
        out_shape=[
            jax.ShapeDtypeStruct((EPAD, D), jnp.bfloat16),
            jax.ShapeDtypeStruct((EPAD, 8), jnp.float32),
        ],
    )(g0, g1, vmask)


def _k4_body(qlo_s, gates_s,
             an, bl, a2s, b2r, gsegl, gsegn, sellast, exn, ohr, colvalid,
             sumsel, rsum_o, csum_o,
             colmin, gmin, m2scr):
    i = pl.program_id(0)
    j = pl.program_id(1)

    @pl.when((i == 0) & (j == 0))
    def _init():
        for c in range(RPAD // 144):
            colmin[pl.ds(c * 144, 144), :] = jnp.full((144, EPAD), BIG,
                                                      jnp.float32)
        rsum_o[...] = jnp.zeros((RPAD, L), jnp.float32)

    dot = _dot_t(an[...], bl[...])
    t = dot + a2s[...][:, 0:1] + b2r[...][0]

    @pl.when(j == 0)
    def _gm0():
        gmin[...] = t

    @pl.when(j > 0)
    def _gmacc():
        gmin[...] = jnp.minimum(gmin[...], t)

    @pl.when(j == NJ - 1)
    def _row_finish():
        m = gmin[...]
        ls = gsegl[...][0:1, :]
        lane = lax.broadcasted_iota(jnp.int32, (1, NG), 1)
        for s in range(10):
            d = 1 << s
            pred = gates_s[0] > d

            @pl.when(pred)
            def _step(d=d):
                gm = gmin[...]
                sh = jnp.roll(gm, d, axis=1)
                shs = jnp.roll(ls, d, axis=1)
                ok = (shs == ls) & (lane >= d)
                gmin[...] = jnp.where(ok, jnp.minimum(gm, sh), gm)

        rowmin = _split_dot(gmin[...], sellast[...])
        rv = -jnp.sqrt(jnp.maximum(rowmin, 1e-12))
        c = _split_dot_l(ohr[...][0], rv)
        q0 = qlo_s[i] * 8
        cur = rsum_o[pl.ds(q0, QW), :]
        rsum_o[pl.ds(q0, QW), :] = cur + c

    g2 = jnp.min(t.reshape(GPT_I, PAD, TJ), axis=1)
    m2scr[...] = g2
    ns = gsegn[...][0][:, 0:1]
    sub = lax.broadcasted_iota(jnp.int32, (GPT_I, 1), 0)
    for s in range(5):
        d = 1 << s

        @pl.when(gates_s[1] > d)
        def _step2(d=d):
            m2 = m2scr[...]
            sh = jnp.roll(m2, d, axis=0)
            shs = jnp.roll(ns, d, axis=0)
            ok = (shs == ns) & (sub >= d)
            m2scr[...] = jnp.where(ok, jnp.minimum(m2, sh), m2)

    ex = exn[...][0]
    e = _split_dot_l(ex, m2scr[...])
    pres = jnp.sum(ex.astype(jnp.float32), axis=1, keepdims=True)
    e = jnp.where(pres > 0.5, e, BIG)
    q0 = qlo_s[i] * 8
    cur = colmin[pl.ds(q0, QW), pl.ds(j * TJ, TJ)]
    colmin[pl.ds(q0, QW), pl.ds(j * TJ, TJ)] = jnp.minimum(cur, e)

    @pl.when((i == NI - 1) & (j == NJ - 1))
    def _epi():
        for c in range(8):
            cm = colmin[pl.ds(c * 64, 64), :]
            cv = jnp.where(cm > 1e29, NEG,
                           -jnp.sqrt(jnp.maximum(cm, 1e-12)))
            cv = cv * colvalid[...][0:1, :]
            gs = jnp.sum(cv.reshape(64, PAD, NG), axis=1)
            csum_o[pl.ds(c * 64, 64), :] = _split_dot(gs, sumsel[...])


def _fused_reduce(an, bl, a2s, b2r, gsegl, gsegn, sellast, exn, ohr,
                  colvalid, sumsel, qlo, gates):
    grid_spec = pltpu.PrefetchScalarGridSpec(
        num_scalar_prefetch=2,
        grid=(NI, NJ),
        in_specs=[
            pl.BlockSpec((TI, D), lambda i, j, *_: (i, 0)),
            pl.BlockSpec((TJ, D), lambda i, j, *_: (j, 0)),
            pl.BlockSpec((TI, 8), lambda i, j, *_: (i, 0)),
            pl.BlockSpec((1, 1, TJ), lambda i, j, *_: (j, 0, 0)),
            pl.BlockSpec((8, NG), lambda i, j, *_: (0, 0)),
            pl.BlockSpec((1, GPT_I, 8), lambda i, j, *_: (i, 0, 0)),
            pl.BlockSpec((NG, L), lambda i, j, *_: (0, 0)),
            pl.BlockSpec((1, QW, GPT_I), lambda i, j, *_: (i, 0, 0)),
            pl.BlockSpec((1, QW, TI), lambda i, j, *_: (i, 0, 0)),
            pl.BlockSpec((8, EPAD), lambda i, j, *_: (0, 0)),
            pl.BlockSpec((NG, L), lambda i, j, *_: (0, 0)),
        ],
        out_specs=[
            pl.BlockSpec((RPAD, L), lambda i, j, *_: (0, 0)),
            pl.BlockSpec((RPAD, L), lambda i, j, *_: (0, 0)),
        ],
        scratch_shapes=[
            pltpu.VMEM((RPAD, EPAD), jnp.float32),
            pltpu.VMEM((TI, NG), jnp.float32),
            pltpu.VMEM((GPT_I, TJ), jnp.float32),
        ],
    )
    return pl.pallas_call(
        _k4_body,
        grid_spec=grid_spec,
        out_shape=[
            jax.ShapeDtypeStruct((RPAD, L), jnp.float32),
            jax.ShapeDtypeStruct((RPAD, L), jnp.float32),
        ],
    )(qlo, gates, an, bl, a2s, b2r, gsegl, gsegn, sellast, exn, ohr,
      colvalid, sumsel)


def _topk_softmax(x):
    idx = lax.broadcasted_iota(jnp.int32, x.shape, 1)
    kept = jnp.zeros(x.shape, jnp.bool_)
    xw = x
    for _ in range(K_TOP):
        mx = jnp.max(xw, axis=1, keepdims=True)
        cand = jnp.where(xw == mx, idx, jnp.int32(2 ** 30))
        amin = jnp.min(cand, axis=1, keepdims=True)
        pick = idx == amin
        kept = kept | pick
        xw = jnp.where(pick, -jnp.inf, xw)
    mx = jnp.max(jnp.where(kept, x, -jnp.inf), axis=1, keepdims=True)
    p = jnp.where(kept, jnp.exp(x - mx), 0.0)
    return p / jnp.sum(p, axis=1, keepdims=True)


def _k5_body(hn, hlt, rsum, csum, cntn, cntl, out):
    hn_ = hn[...]
    hlt_ = hlt[...]
    n2 = jnp.sum(hn_ * hn_, axis=1, keepdims=True)
    l2 = jnp.sum(hlt_ * hlt_, axis=0, keepdims=True)
    ab = _dot(hn_.astype(jnp.bfloat16), hlt_.astype(jnp.bfloat16))
    d2 = n2 + l2 - 2.0 * ab
    node_score = _topk_softmax(-jnp.sqrt(jnp.clip(d2, 1e-12, None)))

    cn = cntn[...][:, 0:1]
    cl = cntl[...][0:1, :]
    rmean = rsum[...] / jnp.maximum(cn, 1.0)
    rmean = jnp.where(cl == 0.0, jnp.where(cn > 0.0, NEG, 0.0), rmean)
    cmean = csum[...] / jnp.maximum(cl, 1.0)
    neigh = _topk_softmax(0.5 * (rmean + cmean))
    out[...] = ALPHA * node_score + (1.0 - ALPHA) * neigh


def _final_stage(hn, hlt, rsum, csum, cntn, cntl):
    return pl.pallas_call(
        _k5_body,
        out_shape=jax.ShapeDtypeStruct((Q, L), jnp.float32),
    )(hn, hlt, rsum, csum, cntn, cntl)


def _padded_layout(seg, nseg):
    gids = jnp.arange(NG, dtype=jnp.int32)
    cnt = jnp.zeros((nseg,), jnp.int32).at[seg].add(1)
    gq = (cnt + PAD - 1) // PAD
    gend = jnp.cumsum(gq).astype(jnp.int32)
    goff = gend - gq
    gseg = jnp.searchsorted(gend, gids, side="right").astype(jnp.int32)
    gvalid = gseg < nseg
    gseg_c = jnp.where(gvalid, gseg, nseg)
    soff = (jnp.cumsum(cnt) - cnt).astype(jnp.int32)
    slot_g = jnp.arange(EPAD, dtype=jnp.int32) // PAD
    slot_r = jnp.arange(EPAD, dtype=jnp.int32) % PAD
    sseg = gseg_c[slot_g]
    sseg_cl = jnp.clip(sseg, 0, nseg - 1)
    rank = (slot_g - goff[sseg_cl]) * PAD + slot_r
    valid = gvalid[slot_g] & (rank < cnt[sseg_cl])
    edge = jnp.where(valid, soff[sseg_cl] + rank, 0)
    return cnt, gq, gseg_c, sseg, valid, edge, goff


def kernel(x, W, node_idx, label_idx, node_edge, label_edge, node_seg,
           label_seg):
    f32 = jnp.float32
    h = _matmul_h(x.astype(jnp.bfloat16), W.astype(jnp.bfloat16))
    htab = jnp.concatenate([h, jnp.zeros((8, NHID), f32)], axis=0)

    cnt_n, gq_n, gseg_n, sseg_n, valid_n, edge_n, _ = _padded_layout(
        node_seg, Q)
    cnt_l, gq_l, gseg_l, sseg_l, valid_l, edge_l, goff_l = _padded_layout(
        label_seg, L)

    zrow = jnp.int32(N_NODES)
    sarr = jnp.arange(EPAD, dtype=jnp.int32)
    perm = (sarr % NG) * PAD + sarr // NG
    valid_l = valid_l[perm]
    edge_l = edge_l[perm]
    idx_n0 = jnp.where(valid_n, node_edge[0][edge_n], zrow)
    idx_n1 = jnp.where(valid_n, node_edge[1][edge_n], zrow)
    idx_l0 = jnp.where(valid_l, label_edge[0][edge_l], zrow)
    idx_l1 = jnp.where(valid_l, label_edge[1][edge_l], zrow)
    allidx = jnp.concatenate([idx_n0, idx_n1, idx_l0, idx_l1,
                              node_idx.astype(jnp.int32),
                              label_idx.astype(jnp.int32)]).astype(jnp.int32)

    gh = _sc_gather(htab, allidx)
    g0, g1 = gh[0:EPAD], gh[EPAD:2 * EPAD]
    g2, g3 = gh[2 * EPAD:3 * EPAD], gh[3 * EPAD:4 * EPAD]
    hn = gh[4 * EPAD:4 * EPAD + Q]
    hl = gh[4 * EPAD + Q:4 * EPAD + Q + L]

    vn8 = jnp.broadcast_to(valid_n.astype(f32)[:, None], (EPAD, 8))
    vl8 = jnp.broadcast_to(valid_l.astype(f32)[:, None], (EPAD, 8))
    efn_b, a2s = _edge_prep(g0, g1, vn8)
    efl_b, b2s = _edge_prep(g2, g3, vl8)
    an = (efn_b * jnp.asarray(-2.0, jnp.bfloat16))
    b2r = b2s[:, 0].reshape(NJ, 1, TJ)

    iarr = jnp.arange(NI, dtype=jnp.int32)
    qlo8 = jnp.clip(sseg_n[iarr * TI], 0, Q - 1) // 8
    qlo = 8 * qlo8
    gates = jnp.stack([jnp.max(gq_l), jnp.max(gq_n)]).astype(jnp.int32)

    gsegl_t = jnp.broadcast_to(gseg_l[None, :], (8, NG))
    gsegn_t = jnp.broadcast_to(gseg_n.reshape(NI, GPT_I)[:, :, None],
                               (NI, GPT_I, 8))

    has_l = cnt_l > 0
    glast = jnp.where(has_l, goff_l + gq_l - 1, -1)
    gidv = jnp.arange(NG, dtype=jnp.int32)
    sellast = (gidv[:, None] == glast[None, :]).astype(jnp.bfloat16)
    sumsel = (gseg_l[:, None] == jnp.arange(L, dtype=jnp.int32)[None, :]
              ).astype(jnp.bfloat16)

    gseg_next = jnp.concatenate([gseg_n[1:], jnp.full((1,), Q, jnp.int32)])
    lastin = ((gidv % GPT_I) == GPT_I - 1) | (gseg_next != gseg_n)
    qr = jnp.arange(QW, dtype=jnp.int32)
    gsegn2 = gseg_n.reshape(NI, 1, GPT_I)
    exn = ((gsegn2 == qlo[:, None, None] + qr[None, :, None])
           & lastin.reshape(NI, 1, GPT_I)).astype(jnp.bfloat16)
    ssegn2 = sseg_n.reshape(NI, 1, TI)
    ohr = ((ssegn2 == qlo[:, None, None] + qr[None, :, None])
           & valid_n.reshape(NI, 1, TI)).astype(jnp.bfloat16)

    colvalid = jnp.broadcast_to(valid_l.astype(f32)[None, :], (8, EPAD))

    rsum, csum = _fused_reduce(an, efl_b, a2s, b2r, gsegl_t, gsegn_t,
                               sellast, exn, ohr, colvalid, sumsel,
                               qlo8, gates)

    cntn8 = jnp.broadcast_to(cnt_n.astype(f32)[:, None], (Q, 8))
    cntl8 = jnp.broadcast_to(cnt_l.astype(f32)[None, :], (8, L))
    return _final_stage(hn, hl.T, rsum[:Q], csum[:Q], cntn8, cntl8)
